# Initial kernel scaffold; baseline (speedup 1.0000x reference)
import jax
import jax.numpy as jnp
from jax.experimental import pallas as pl


def kernel(boxes, scores):
    raise NotImplementedError("write your pallas kernel here")


# speedup vs baseline: 122.6168x; 122.6168x over previous
import jax
import jax.numpy as jnp
from jax import lax
from jax.experimental import pallas as pl

_N = 5000
_BLK = 128
_NB = 40
_NP = _NB * _BLK
_THR = 0.5


def _nms_body(xr, yr, Xr, Yr, xc, yc, Xc, Yc, keep_ref):
    q_lt_p = (lax.broadcasted_iota(jnp.int32, (_BLK, _BLK), 0)
              < lax.broadcasted_iota(jnp.int32, (_BLK, _BLK), 1))

    def block_step(b, carry):
        rx1 = xr[pl.ds(b, 1), :]
        ry1 = yr[pl.ds(b, 1), :]
        rx2 = Xr[pl.ds(b, 1), :]
        ry2 = Yr[pl.ds(b, 1), :]
        r_area = (rx2 - rx1) * (ry2 - ry1)

        def iou_vs(off):
            cx1 = xc[pl.ds(off, _BLK), :]
            cy1 = yc[pl.ds(off, _BLK), :]
            cx2 = Xc[pl.ds(off, _BLK), :]
            cy2 = Yc[pl.ds(off, _BLK), :]
            c_area = (cx2 - cx1) * (cy2 - cy1)
            xx1 = jnp.maximum(cx1, rx1)
            yy1 = jnp.maximum(cy1, ry1)
            xx2 = jnp.minimum(cx2, rx2)
            yy2 = jnp.minimum(cy2, ry2)
            w = jnp.maximum(xx2 - xx1, 0.0)
            h = jnp.maximum(yy2 - yy1, 0.0)
            inter = w * h
            return inter / (c_area + r_area - inter + 1e-9)

        def cross(j, alive):
            adj = (iou_vs(j * _BLK) > _THR).astype(jnp.float32)
            kprev = keep_ref[pl.ds(j, 1), :]
            supp = lax.dot_general(kprev, adj, (((1,), (0,)), ((), ())),
                                   preferred_element_type=jnp.float32)
            return jnp.where(supp > 0.0, 0.0, alive)

        base = lax.fori_loop(0, b, cross, jnp.ones((1, _BLK), jnp.float32))

        adj_self = jnp.where((iou_vs(b * _BLK) > _THR) & q_lt_p, 1.0, 0.0)

        def fix_body(c):
            alive, _ = c
            supp = lax.dot_general(alive, adj_self, (((1,), (0,)), ((), ())),
                                   preferred_element_type=jnp.float32)
            new = jnp.where(supp > 0.0, 0.0, base)
            return new, jnp.any(new != alive)

        alive, _ = lax.while_loop(lambda c: c[1], fix_body, (base, True))
        keep_ref[pl.ds(b, 1), :] = alive
        return carry

    lax.fori_loop(0, _NB, block_step, 0)


def _nms_sorted_keep(bp):
    x, y, X, Y = bp[:, 0], bp[:, 1], bp[:, 2], bp[:, 3]
    args = (x.reshape(_NB, _BLK), y.reshape(_NB, _BLK),
            X.reshape(_NB, _BLK), Y.reshape(_NB, _BLK),
            x.reshape(_NP, 1), y.reshape(_NP, 1),
            X.reshape(_NP, 1), Y.reshape(_NP, 1))
    keep = pl.pallas_call(
        _nms_body,
        out_shape=jax.ShapeDtypeStruct((_NB, _BLK), jnp.float32),
    )(*args)
    return keep.reshape(_NP)


def kernel(boxes, scores):
    order = jnp.argsort(-scores)
    bs = boxes[order]
    bp = jnp.pad(bs, ((0, _NP - _N), (0, 0)))
    keep_sorted = _nms_sorted_keep(bp)[:_N]
    mask = jnp.zeros((_N,), jnp.float32).at[order].set(keep_sorted)
    out = jnp.concatenate([boxes * mask[:, None], (scores * mask)[:, None]],
                          axis=1)
    return out


# speedup vs baseline: 287.0773x; 2.3413x over previous
import jax
import jax.numpy as jnp
from jax import lax
from jax.experimental import pallas as pl

_N = 5000
_BLK = 512
_NB = 10
_NP = _NB * _BLK
_THR = 0.5


def _nms_body(xr, yr, Xr, Yr, xc, yc, Xc, Yc, keep_ref):
    q_lt_p = (lax.broadcasted_iota(jnp.int32, (_BLK, _BLK), 0)
              < lax.broadcasted_iota(jnp.int32, (_BLK, _BLK), 1))

    def block_step(b, carry):
        rx1 = xr[pl.ds(b, 1), :]
        ry1 = yr[pl.ds(b, 1), :]
        rx2 = Xr[pl.ds(b, 1), :]
        ry2 = Yr[pl.ds(b, 1), :]
        r_area = (rx2 - rx1) * (ry2 - ry1)

        def iou_vs(off):
            cx1 = xc[pl.ds(off, _BLK), :]
            cy1 = yc[pl.ds(off, _BLK), :]
            cx2 = Xc[pl.ds(off, _BLK), :]
            cy2 = Yc[pl.ds(off, _BLK), :]
            c_area = (cx2 - cx1) * (cy2 - cy1)
            xx1 = jnp.maximum(cx1, rx1)
            yy1 = jnp.maximum(cy1, ry1)
            xx2 = jnp.minimum(cx2, rx2)
            yy2 = jnp.minimum(cy2, ry2)
            w = jnp.maximum(xx2 - xx1, 0.0)
            h = jnp.maximum(yy2 - yy1, 0.0)
            inter = w * h
            return inter / (c_area + r_area - inter + 1e-9)

        def cross(j, alive):
            adj = (iou_vs(j * _BLK) > _THR).astype(jnp.float32)
            kprev = keep_ref[pl.ds(j, 1), :]
            supp = lax.dot_general(kprev, adj, (((1,), (0,)), ((), ())),
                                   preferred_element_type=jnp.float32)
            return jnp.where(supp > 0.0, 0.0, alive)

        base = lax.fori_loop(0, b, cross, jnp.ones((1, _BLK), jnp.float32))

        adj_self = jnp.where((iou_vs(b * _BLK) > _THR) & q_lt_p, 1.0, 0.0)

        def fix_body(c):
            alive, _ = c
            supp = lax.dot_general(alive, adj_self, (((1,), (0,)), ((), ())),
                                   preferred_element_type=jnp.float32)
            new = jnp.where(supp > 0.0, 0.0, base)
            return new, jnp.any(new != alive)

        alive, _ = lax.while_loop(lambda c: c[1], fix_body, (base, True))
        keep_ref[pl.ds(b, 1), :] = alive
        return carry

    lax.fori_loop(0, _NB, block_step, 0)


def _nms_sorted_keep(bp):
    x, y, X, Y = bp[:, 0], bp[:, 1], bp[:, 2], bp[:, 3]
    args = (x.reshape(_NB, _BLK), y.reshape(_NB, _BLK),
            X.reshape(_NB, _BLK), Y.reshape(_NB, _BLK),
            x.reshape(_NP, 1), y.reshape(_NP, 1),
            X.reshape(_NP, 1), Y.reshape(_NP, 1))
    keep = pl.pallas_call(
        _nms_body,
        out_shape=jax.ShapeDtypeStruct((_NB, _BLK), jnp.float32),
    )(*args)
    return keep.reshape(_NP)


def kernel(boxes, scores):
    order = jnp.argsort(-scores)
    bs = boxes[order]
    bp = jnp.pad(bs, ((0, _NP - _N), (0, 0)))
    keep_sorted = _nms_sorted_keep(bp)[:_N]
    mask = jnp.zeros((_N,), jnp.float32).at[order].set(keep_sorted)
    out = jnp.concatenate([boxes * mask[:, None], (scores * mask)[:, None]],
                          axis=1)
    return out
